# baseline (speedup 1.0000x reference)
import jax
import jax.numpy as jnp
from jax import lax
from jax.experimental import pallas as pl
from jax.experimental.pallas import tpu as pltpu
from jax.experimental.pallas import tpu_sc as plsc

L = 3276800
V = 5
D = 6
DP = 16
NC = 2
NS = 16
NW = NC * NS
PER = L // NW
CH = 2048
GB = 128
NGB = CH // GB
NCH = PER // CH
LANES = 16


def _sc_body(ids_hbm, table_hbm, emb_hbm, cnt_hbm, ids_v, rows_v, cnt_v, sem):
    cid = lax.axis_index("c")
    sid = lax.axis_index("s")
    wid = sid * NC + cid
    base = wid * PER

    zero = jnp.zeros((LANES,), jnp.float32)
    one = jnp.ones((LANES,), jnp.float32)

    def chunk(c, carry):
        a1, a2, a3, a4 = carry
        off = base + c * CH
        pltpu.sync_copy(ids_hbm.at[pl.ds(off // GB, NGB)], ids_v)
        copies = []
        for j in range(NGB):
            copies.append(
                pltpu.async_copy(
                    table_hbm.at[ids_v.at[j]],
                    rows_v.at[pl.ds(j * GB, GB)],
                    sem,
                )
            )
        def hist(r, acc):
            h1, h2, h3, h4 = acc
            x = ids_v[r // (GB // LANES), pl.ds((r % (GB // LANES)) * LANES, LANES)]
            h1 = h1 + jnp.where(x == 1, one, zero)
            h2 = h2 + jnp.where(x == 2, one, zero)
            h3 = h3 + jnp.where(x == 3, one, zero)
            h4 = h4 + jnp.where(x == 4, one, zero)
            return (h1, h2, h3, h4)

        a1, a2, a3, a4 = lax.fori_loop(0, CH // LANES, hist, (a1, a2, a3, a4))
        for cp in copies:
            cp.wait()
        pltpu.sync_copy(rows_v.at[:, pl.ds(0, D)], emb_hbm.at[pl.ds(off, CH)])
        return (a1, a2, a3, a4)

    a1, a2, a3, a4 = lax.fori_loop(0, NCH, chunk, (zero, zero, zero, zero))
    cnt_v[pl.ds(0, LANES)] = a1
    cnt_v[pl.ds(16, LANES)] = a2
    cnt_v[pl.ds(32, LANES)] = a3
    cnt_v[pl.ds(48, LANES)] = a4
    pltpu.sync_copy(cnt_v, cnt_hbm.at[wid])


_sc_call = pl.kernel(
    _sc_body,
    out_type=(
        jax.ShapeDtypeStruct((L, D), jnp.float32),
        jax.ShapeDtypeStruct((NW, 4 * LANES), jnp.float32),
    ),
    mesh=plsc.VectorSubcoreMesh(
        core_axis_name="c", subcore_axis_name="s", num_cores=NC, num_subcores=NS
    ),
    scratch_types=[
        pltpu.VMEM((NGB, GB), jnp.int32),
        pltpu.VMEM((CH, DP), jnp.float32),
        pltpu.VMEM((4 * LANES,), jnp.float32),
        pltpu.SemaphoreType.DMA,
    ],
    compiler_params=pltpu.CompilerParams(use_tc_tiling_on_sc=False),
)


def _finale_body(cnt_ref, table_ref, fcw_ref, fcb_ref, out_ref):
    s = jnp.sum(cnt_ref[...], axis=0, keepdims=True)
    rows = lax.broadcasted_iota(jnp.int32, (4 * LANES, V), 0)
    cols = lax.broadcasted_iota(jnp.int32, (4 * LANES, V), 1)
    sel = jnp.where(rows // LANES + 1 == cols, 1.0, 0.0)
    counts = jnp.dot(s, sel, preferred_element_type=jnp.float32)
    c0 = jnp.float32(L) - jnp.sum(counts)
    col1 = lax.broadcasted_iota(jnp.int32, (1, V), 1)
    counts = counts + jnp.where(col1 == 0, c0, 0.0)
    pooled = jnp.dot(counts, table_ref[...], preferred_element_type=jnp.float32)
    pooled = pooled * jnp.float32(1.0 / L)
    logits = (
        jnp.dot(pooled, fcw_ref[...].T, preferred_element_type=jnp.float32)
        + fcb_ref[...]
    )
    out_ref[...] = logits


_finale_call = pl.pallas_call(
    _finale_body,
    out_shape=jax.ShapeDtypeStruct((1, 2), jnp.float32),
)


@jax.jit
def kernel(x_ids, table, fc_w, fc_b):
    table_p = jnp.pad(table, ((0, 0), (0, DP - D)))
    ids2d = x_ids.reshape(L // GB, GB)
    emb, cnt = _sc_call(ids2d, table_p)
    logits = _finale_call(cnt, table, fc_w, fc_b.reshape(1, 2)).reshape(2)
    return logits, emb


# speedup vs baseline: 7.1207x; 7.1207x over previous
import jax
import jax.numpy as jnp
from jax import lax
from jax.experimental import pallas as pl
from jax.experimental.pallas import tpu as pltpu
from jax.experimental.pallas import tpu_sc as plsc

L = 3276800
V = 5
D = 6
DP = 16
NC = 2
NS = 16
NW = NC * NS
PER = L // NW
CH = 2048
NCH = PER // CH
LANES = 16
GPC = CH // LANES


def _sc_body(ids_hbm, table_hbm, emb_hbm, cnt_hbm, ids_v, out_v, tvm, cnt_v, sem):
    cid = lax.axis_index("c")
    sid = lax.axis_index("s")
    wid = sid * NC + cid
    base = wid * PER

    pltpu.sync_copy(table_hbm, tvm)
    cnt_v[pl.ds(0, LANES)] = jnp.zeros((LANES,), jnp.float32)

    ones = jnp.ones((LANES,), jnp.float32)
    lane = lax.iota(jnp.int32, LANES)
    six = lane * 6
    jconst = [jnp.full((LANES,), j, jnp.int32) for j in range(D)]
    sixp = [six + j for j in range(D)]

    def chunk(c, _):
        off = base + c * CH
        pltpu.sync_copy(ids_hbm.at[pl.ds(off, CH)], ids_v)

        def group(g, _):
            idv = ids_v[pl.ds(g * LANES, LANES)]
            plsc.addupdate_scatter(cnt_v, [idv], ones)
            outs = out_v.at[pl.ds(g * (LANES * D), LANES * D)]
            for j in range(D):
                vals = plsc.load_gather(tvm, [idv, jconst[j]])
                plsc.store_scatter(outs, [sixp[j]], vals)
            return 0

        lax.fori_loop(0, GPC, group, 0, unroll=4)
        pltpu.sync_copy(out_v, emb_hbm.at[pl.ds(off * D, CH * D)])
        return 0

    lax.fori_loop(0, NCH, chunk, 0)
    pltpu.sync_copy(cnt_v, cnt_hbm.at[wid])


_sc_call = pl.kernel(
    _sc_body,
    out_type=(
        jax.ShapeDtypeStruct((L * D,), jnp.float32),
        jax.ShapeDtypeStruct((NW, LANES), jnp.float32),
    ),
    mesh=plsc.VectorSubcoreMesh(
        core_axis_name="c", subcore_axis_name="s", num_cores=NC, num_subcores=NS
    ),
    scratch_types=[
        pltpu.VMEM((CH,), jnp.int32),
        pltpu.VMEM((CH * D,), jnp.float32),
        pltpu.VMEM((V, DP), jnp.float32),
        pltpu.VMEM((LANES,), jnp.float32),
        pltpu.SemaphoreType.DMA,
    ],
    compiler_params=pltpu.CompilerParams(
        use_tc_tiling_on_sc=False, needs_layout_passes=False
    ),
)


def _finale_body(cnt_ref, table_ref, fcw_ref, fcb_ref, out_ref):
    s = jnp.sum(cnt_ref[...], axis=0, keepdims=True)
    rows = lax.broadcasted_iota(jnp.int32, (LANES, V), 0)
    cols = lax.broadcasted_iota(jnp.int32, (LANES, V), 1)
    sel = jnp.where(rows == cols, 1.0, 0.0)
    counts = jnp.dot(s, sel, preferred_element_type=jnp.float32)
    pooled = jnp.dot(counts, table_ref[...], preferred_element_type=jnp.float32)
    pooled = pooled * jnp.float32(1.0 / L)
    logits = (
        jnp.dot(pooled, fcw_ref[...].T, preferred_element_type=jnp.float32)
        + fcb_ref[...]
    )
    out_ref[...] = logits


_finale_call = pl.pallas_call(
    _finale_body,
    out_shape=jax.ShapeDtypeStruct((1, 2), jnp.float32),
)


@jax.jit
def kernel(x_ids, table, fc_w, fc_b):
    table_p = jnp.pad(table, ((0, 0), (0, DP - D)))
    emb_flat, cnt = _sc_call(x_ids, table_p)
    logits = _finale_call(cnt, table, fc_w, fc_b.reshape(1, 2)).reshape(2)
    return logits, emb_flat.reshape(L, D)
